# Initial kernel scaffold; baseline (speedup 1.0000x reference)
import jax
import jax.numpy as jnp
from jax.experimental import pallas as pl


def kernel(x, weight, bias, gamma, beta):
    raise NotImplementedError("write your pallas kernel here")


# speedup vs baseline: 1.5833x; 1.5833x over previous
import functools

import jax
import jax.numpy as jnp
from jax import lax
from jax.experimental import pallas as pl
from jax.experimental.pallas import tpu as pltpu

_EPS = 1e-5


def _round_up(x, m):
    return (x + m - 1) // m * m


def _fused_kernel(xa_ref, xb_ref, wt_ref, mask_ref, g_ref, b_ref,
                  out_ref, sum_ref, ssq_ref,
                  *, k, Wp, PQ, PT, Cout, cnt):
    t = pl.program_id(1)

    win = jnp.concatenate([xa_ref[0], xb_ref[0]], axis=-1)

    acc = jnp.zeros((Cout, PQ), jnp.float32)
    for tap in range(k * k):
        off = (tap // k) * Wp + (tap % k)
        acc = acc + jnp.dot(wt_ref[tap], win[:, off:off + PQ],
                            preferred_element_type=jnp.float32)

    @pl.when(t == 0)
    def _():
        sum_ref[...] = jnp.zeros_like(sum_ref)
        ssq_ref[...] = jnp.zeros_like(ssq_ref)

    for tt in range(PT):
        @pl.when(t == tt)
        def _(tt=tt):
            out_ref[0, :, tt * PQ:(tt + 1) * PQ] = acc

    am = acc * mask_ref[...]
    sum_ref[...] += jnp.sum(am, axis=-1, keepdims=True)
    ssq_ref[...] += jnp.sum(am * am, axis=-1, keepdims=True)

    @pl.when(t == PT - 1)
    def _():
        mean = sum_ref[...] / cnt
        var = jnp.maximum(ssq_ref[...] / cnt - mean * mean, 0.0)
        scale = g_ref[...] * lax.rsqrt(var + _EPS)
        shift = b_ref[...] - mean * scale
        y = out_ref[0]
        out_ref[0] = jnp.maximum(y * scale + shift, 0.0)


def kernel(x, weight, bias, gamma, beta):
    del bias
    N, Cin, H, W = x.shape
    Cout = weight.shape[0]
    k = 3
    p = k // 2

    x_pad = jnp.pad(x, ((0, 0), (0, 0), (p, p), (p, p)), mode="reflect")
    Hp, Wp = H + 2 * p, W + 2 * p
    Ho, Wo = H, W

    PV_raw = Ho * Wp
    OVR = (k - 1) * Wp + (k - 1)
    PV = _round_up(PV_raw, 128)
    nl = PV // 128
    PT = 1
    for cand in (4, 3, 2):
        if nl % cand == 0 and (PV // cand) >= max(OVR, 256):
            PT = cand
            break
    PQ = PV // PT
    L = (PT + 1) * PQ

    xf = x_pad.reshape(N, Cin, Hp * Wp)
    xf = jnp.pad(xf, ((0, 0), (0, 0), (0, L - Hp * Wp))).astype(jnp.bfloat16)

    wt = jnp.transpose(weight, (2, 3, 0, 1)).reshape(k * k, Cout, Cin)
    wt = wt.astype(jnp.bfloat16)

    q = jnp.arange(PV, dtype=jnp.int32)
    mask = ((q < PV_raw) & ((q % Wp) < Wo)).astype(jnp.float32)[None, :]

    g2 = gamma.astype(jnp.float32).reshape(Cout, 1)
    b2 = beta.astype(jnp.float32).reshape(Cout, 1)

    kern = functools.partial(_fused_kernel, k=k, Wp=Wp, PQ=PQ, PT=PT,
                             Cout=Cout, cnt=float(Ho * Wo))
    y = pl.pallas_call(
        kern,
        out_shape=jax.ShapeDtypeStruct((N, Cout, PV), jnp.float32),
        grid_spec=pltpu.PrefetchScalarGridSpec(
            num_scalar_prefetch=0,
            grid=(N, PT),
            in_specs=[
                pl.BlockSpec((1, Cin, PQ), lambda n, t: (n, 0, t)),
                pl.BlockSpec((1, Cin, PQ), lambda n, t: (n, 0, t + 1)),
                pl.BlockSpec((k * k, Cout, Cin), lambda n, t: (0, 0, 0)),
                pl.BlockSpec((1, PQ), lambda n, t: (0, t)),
                pl.BlockSpec((Cout, 1), lambda n, t: (0, 0)),
                pl.BlockSpec((Cout, 1), lambda n, t: (0, 0)),
            ],
            out_specs=pl.BlockSpec((1, Cout, PV), lambda n, t: (n, 0, 0)),
            scratch_shapes=[
                pltpu.VMEM((Cout, 1), jnp.float32),
                pltpu.VMEM((Cout, 1), jnp.float32),
            ],
        ),
        compiler_params=pltpu.CompilerParams(
            dimension_semantics=("parallel", "arbitrary"),
            vmem_limit_bytes=64 * 1024 * 1024),
    )(xf, xf, wt, mask, g2, b2)

    out = y[:, :, :PV_raw].reshape(N, Cout, Ho, Wp)[:, :, :, :Wo]
    return out


# speedup vs baseline: 2.0130x; 1.2714x over previous
import functools

import jax
import jax.numpy as jnp
from jax import lax
from jax.experimental import pallas as pl
from jax.experimental.pallas import tpu as pltpu

_EPS = 1e-5


def _round_up(x, m):
    return (x + m - 1) // m * m


def _fused_kernel(x_ref, wt_ref, mask_ref, g_ref, b_ref, out_ref,
                  win_ref, xcol_ref, yacc_ref,
                  *, k, H, W, Cin, Cout, LW):
    Wp = W + 2
    Ho, Wo = H, W
    PV = Ho * Wp
    x = x_ref[0]

    tail = (PV + (k - 1) * Wp + (k - 1)) // 128 * 128
    win_ref[:, tail - 128:] = jnp.zeros((Cin, LW - tail + 128), jnp.bfloat16)

    for r in range(Ho + 2):
        pr = 1 if r == 0 else (H - 2 if r == Ho + 1 else r - 1)
        row = x[:, pr * W:(pr + 1) * W].astype(jnp.bfloat16)
        win_ref[:, r * Wp + 1:r * Wp + 1 + W] = row
        win_ref[:, r * Wp:r * Wp + 1] = row[:, 1:2]
        win_ref[:, r * Wp + 1 + W:r * Wp + 2 + W] = row[:, W - 2:W - 1]

    for t in range(k * k):
        off = (t // k) * Wp + (t % k)
        xcol_ref[t * Cin:(t + 1) * Cin, :] = win_ref[:, off:off + PV]

    yacc_ref[...] = jnp.dot(wt_ref[...], xcol_ref[...],
                            preferred_element_type=jnp.float32)

    am = yacc_ref[...] * mask_ref[...]
    cnt = float(Ho * Wo)
    s = jnp.sum(am, axis=-1, keepdims=True)
    sq = jnp.sum(am * am, axis=-1, keepdims=True)
    mean = s / cnt
    var = jnp.maximum(sq / cnt - mean * mean, 0.0)
    scale = g_ref[...] * lax.rsqrt(var + _EPS)
    shift = b_ref[...] - mean * scale

    for h in range(Ho):
        row = yacc_ref[:, h * Wp:h * Wp + Wo]
        out_ref[0, :, h * Wo:(h + 1) * Wo] = jnp.maximum(
            row * scale + shift, 0.0)


def kernel(x, weight, bias, gamma, beta):
    del bias
    N, Cin, H, W = x.shape
    Cout = weight.shape[0]
    k = 3

    Wp = W + 2
    Ho, Wo = H, W
    PV = Ho * Wp
    OVR = (k - 1) * Wp + (k - 1)
    LW = _round_up(max(PV + OVR, (Ho + 2) * Wp), 128)

    xf = x.reshape(N, Cin, H * W)

    wt = jnp.transpose(weight, (0, 2, 3, 1)).reshape(Cout, k * k * Cin)
    wt = wt.astype(jnp.bfloat16)

    q = jnp.arange(PV, dtype=jnp.int32)
    mask = ((q % Wp) < Wo).astype(jnp.float32)[None, :]

    g2 = gamma.astype(jnp.float32).reshape(Cout, 1)
    b2 = beta.astype(jnp.float32).reshape(Cout, 1)

    kern = functools.partial(_fused_kernel, k=k, H=H, W=W, Cin=Cin,
                             Cout=Cout, LW=LW)
    y = pl.pallas_call(
        kern,
        out_shape=jax.ShapeDtypeStruct((N, Cout, H * W), jnp.float32),
        grid_spec=pltpu.PrefetchScalarGridSpec(
            num_scalar_prefetch=0,
            grid=(N,),
            in_specs=[
                pl.BlockSpec((1, Cin, H * W), lambda n: (n, 0, 0)),
                pl.BlockSpec((Cout, k * k * Cin), lambda n: (0, 0)),
                pl.BlockSpec((1, PV), lambda n: (0, 0)),
                pl.BlockSpec((Cout, 1), lambda n: (0, 0)),
                pl.BlockSpec((Cout, 1), lambda n: (0, 0)),
            ],
            out_specs=pl.BlockSpec((1, Cout, H * W), lambda n: (n, 0, 0)),
            scratch_shapes=[
                pltpu.VMEM((Cin, LW), jnp.bfloat16),
                pltpu.VMEM((k * k * Cin, PV), jnp.bfloat16),
                pltpu.VMEM((Cout, PV), jnp.float32),
            ],
        ),
        compiler_params=pltpu.CompilerParams(
            dimension_semantics=("parallel",),
            vmem_limit_bytes=48 * 1024 * 1024),
    )(xf, wt, mask, g2, b2)

    return y.reshape(N, Cout, H, W)
